# Initial kernel scaffold; baseline (speedup 1.0000x reference)
import jax
import jax.numpy as jnp
from jax.experimental import pallas as pl


def kernel(x, edge_index, W1, b1, W2, b2, W3, b3):
    raise NotImplementedError("write your pallas kernel here")


# speedup vs baseline: 22.4782x; 22.4782x over previous
import functools

import jax
import jax.numpy as jnp
from jax import lax
from jax.experimental import pallas as pl
from jax.experimental.pallas import tpu as pltpu
from jax.experimental.pallas import tpu_sc as plsc

N_NODES = 10000
N_PAD = 10240
N_EDGES = 320000
NC = 2
NS = 16
NW = NC * NS
E_TILE = N_EDGES // NW
CHUNK = 80
NCHUNK = E_TILE // CHUNK
ROWS_TILE = N_PAD // NS

_MESH = plsc.VectorSubcoreMesh(
    core_axis_name="c", subcore_axis_name="s", num_cores=NC, num_subcores=NS)


def _deg_body(dst_hbm, ones_hbm, zeros_hbm, out_hbm, idx_v, ones_v, acc_sh, sem):
    c = lax.axis_index("c")
    s = lax.axis_index("s")
    wid = c * NS + s
    pltpu.sync_copy(zeros_hbm.at[pl.ds(s * ROWS_TILE, ROWS_TILE)],
                    acc_sh.at[pl.ds(s * ROWS_TILE, ROWS_TILE)])
    pltpu.sync_copy(ones_hbm, ones_v)
    pltpu.sync_copy(dst_hbm.at[wid], idx_v)
    plsc.subcore_barrier()

    def chunk(j, carry):
        pltpu.async_copy(ones_v, acc_sh.at[idx_v.at[j]], sem, add=True).wait()
        return carry

    lax.fori_loop(0, NCHUNK, chunk, 0)
    plsc.subcore_barrier()
    pltpu.sync_copy(acc_sh.at[pl.ds(s * ROWS_TILE, ROWS_TILE)],
                    out_hbm.at[c, pl.ds(s * ROWS_TILE, ROWS_TILE)])


def _make_agg_body(width):
    def _agg_body(h_hbm, src_hbm, dst_hbm, zeros_hbm, out_hbm,
                  idxs_v, idxd_v, rows_v, acc_sh, gsem, ssem):
        c = lax.axis_index("c")
        s = lax.axis_index("s")
        wid = c * NS + s
        pltpu.sync_copy(zeros_hbm.at[pl.ds(s * ROWS_TILE, ROWS_TILE)],
                        acc_sh.at[pl.ds(s * ROWS_TILE, ROWS_TILE)])
        pltpu.sync_copy(src_hbm.at[wid], idxs_v)
        pltpu.sync_copy(dst_hbm.at[wid], idxd_v)
        plsc.subcore_barrier()

        def chunk(j, carry):
            pltpu.async_copy(h_hbm.at[idxs_v.at[j]], rows_v, gsem).wait()
            pltpu.async_copy(rows_v, acc_sh.at[idxd_v.at[j]], ssem,
                             add=True).wait()
            return carry

        lax.fori_loop(0, NCHUNK, chunk, 0)
        plsc.subcore_barrier()
        pltpu.sync_copy(acc_sh.at[pl.ds(s * ROWS_TILE, ROWS_TILE)],
                        out_hbm.at[c, pl.ds(s * ROWS_TILE, ROWS_TILE)])

    return _agg_body


def _sc_degree(dst3, ones8, zeros8):
    return pl.kernel(
        _deg_body,
        out_type=jax.ShapeDtypeStruct((NC, N_PAD, 8), jnp.float32),
        mesh=_MESH,
        compiler_params=pltpu.CompilerParams(use_tc_tiling_on_sc=False),
        scratch_types=[
            pltpu.VMEM((NCHUNK, CHUNK), jnp.int32),
            pltpu.VMEM((CHUNK, 8), jnp.float32),
            pltpu.VMEM_SHARED((N_PAD, 8), jnp.float32),
            pltpu.SemaphoreType.DMA,
        ],
    )(dst3, ones8, zeros8)


def _sc_aggregate(h, src3, dst3, zeros, width):
    return pl.kernel(
        _make_agg_body(width),
        out_type=jax.ShapeDtypeStruct((NC, N_PAD, width), jnp.float32),
        mesh=_MESH,
        compiler_params=pltpu.CompilerParams(use_tc_tiling_on_sc=False),
        scratch_types=[
            pltpu.VMEM((NCHUNK, CHUNK), jnp.int32),
            pltpu.VMEM((NCHUNK, CHUNK), jnp.int32),
            pltpu.VMEM((CHUNK, width), jnp.float32),
            pltpu.VMEM_SHARED((N_PAD, width), jnp.float32),
            pltpu.SemaphoreType.DMA,
            pltpu.SemaphoreType.DMA,
        ],
    )(h, src3, dst3, zeros)


_ROWS_BLK = 640


def _tc_first_body(degp_ref, x_ref, w_ref, hp_ref, dis_ref):
    d = degp_ref[...]
    deg = d[0, :, 0:1] + d[1, :, 0:1] + 1.0
    dis = lax.rsqrt(deg)
    h = jnp.dot(x_ref[...], w_ref[...], preferred_element_type=jnp.float32)
    hp_ref[...] = h * dis
    dis_ref[...] = dis


def _tc_first(degp, x, W1):
    n, f = x.shape
    h = W1.shape[1]
    grid = n // _ROWS_BLK
    return pl.pallas_call(
        _tc_first_body,
        grid=(grid,),
        in_specs=[
            pl.BlockSpec((NC, _ROWS_BLK, 8), lambda i: (0, i, 0)),
            pl.BlockSpec((_ROWS_BLK, f), lambda i: (i, 0)),
            pl.BlockSpec((f, h), lambda i: (0, 0)),
        ],
        out_specs=[
            pl.BlockSpec((_ROWS_BLK, h), lambda i: (i, 0)),
            pl.BlockSpec((_ROWS_BLK, 1), lambda i: (i, 0)),
        ],
        out_shape=[
            jax.ShapeDtypeStruct((n, h), jnp.float32),
            jax.ShapeDtypeStruct((n, 1), jnp.float32),
        ],
    )(degp, x, W1)


def _tc_mid_body(p_ref, hp_ref, dis_ref, b_ref, w_ref, out_ref):
    p = p_ref[...]
    dis = dis_ref[...]
    z = (p[0] + p[1] + hp_ref[...]) * dis + b_ref[...]
    hact = jnp.where(z >= 0.0, z, 0.01 * z)
    out_ref[...] = jnp.dot(hact, w_ref[...],
                           preferred_element_type=jnp.float32) * dis


def _tc_mid(part, hp, dis, b, W):
    n, h = hp.shape
    h2 = W.shape[1]
    grid = n // _ROWS_BLK
    return pl.pallas_call(
        _tc_mid_body,
        grid=(grid,),
        in_specs=[
            pl.BlockSpec((NC, _ROWS_BLK, h), lambda i: (0, i, 0)),
            pl.BlockSpec((_ROWS_BLK, h), lambda i: (i, 0)),
            pl.BlockSpec((_ROWS_BLK, 1), lambda i: (i, 0)),
            pl.BlockSpec((1, h), lambda i: (0, 0)),
            pl.BlockSpec((h, h2), lambda i: (0, 0)),
        ],
        out_specs=pl.BlockSpec((_ROWS_BLK, h2), lambda i: (i, 0)),
        out_shape=jax.ShapeDtypeStruct((n, h2), jnp.float32),
    )(part, hp, dis, b, W)


def _tc_last_body(p_ref, hp_ref, dis_ref, b_ref, out_ref):
    p = p_ref[...]
    z = (p[0, :, 0:1] + p[1, :, 0:1] + hp_ref[:, 0:1]) * dis_ref[...]
    out_ref[...] = z + b_ref[...]


def _tc_last(part8, hp8, dis, b3):
    n = hp8.shape[0]
    grid = n // _ROWS_BLK
    return pl.pallas_call(
        _tc_last_body,
        grid=(grid,),
        in_specs=[
            pl.BlockSpec((NC, _ROWS_BLK, 8), lambda i: (0, i, 0)),
            pl.BlockSpec((_ROWS_BLK, 8), lambda i: (i, 0)),
            pl.BlockSpec((_ROWS_BLK, 1), lambda i: (i, 0)),
            pl.BlockSpec((1, 1), lambda i: (0, 0)),
        ],
        out_specs=pl.BlockSpec((_ROWS_BLK, 1), lambda i: (i, 0)),
        out_shape=jax.ShapeDtypeStruct((n, 1), jnp.float32),
    )(part8, hp8, dis, b3)


def kernel(x, edge_index, W1, b1, W2, b2, W3, b3):
    n, f = x.shape
    assert n == N_NODES and edge_index.shape[1] == N_EDGES
    src3 = edge_index[0].astype(jnp.int32).reshape(NW, NCHUNK, CHUNK)
    dst3 = edge_index[1].astype(jnp.int32).reshape(NW, NCHUNK, CHUNK)
    ones8 = jnp.ones((CHUNK, 8), jnp.float32)
    zeros8 = jnp.zeros((N_PAD, 8), jnp.float32)
    zeros64 = jnp.zeros((N_PAD, 64), jnp.float32)
    w3w = jnp.broadcast_to(W3, (W3.shape[0], 8))
    xp = jnp.pad(x, ((0, N_PAD - n), (0, 0)))

    degp = _sc_degree(dst3, ones8, zeros8)
    h1p, dis = _tc_first(degp, xp, W1)
    part1 = _sc_aggregate(h1p, src3, dst3, zeros64, 64)
    h2p = _tc_mid(part1, h1p, dis, b1.reshape(1, -1), W2)
    part2 = _sc_aggregate(h2p, src3, dst3, zeros64, 64)
    h3p = _tc_mid(part2, h2p, dis, b2.reshape(1, -1), w3w)
    part3 = _sc_aggregate(h3p, src3, dst3, zeros8, 8)
    return _tc_last(part3, h3p, dis, b3.reshape(1, 1))[:n]
